# Initial kernel scaffold; baseline (speedup 1.0000x reference)
import functools

import jax
import jax.numpy as jnp
from jax import lax
from jax.experimental import pallas as pl
from jax.experimental.pallas import tpu as pltpu

N_ITER = 2
DT = 0.01
COORDS_WEIGHT = 1.0

BE = 512
BN = 1000

_HIGH = jax.lax.Precision.HIGHEST


def _silu(x):
    return x * jax.nn.sigmoid(x)


def _dot(a, b):
    return jax.lax.dot_general(a, b, (((1,), (0,)), ((), ())),
                               precision=_HIGH,
                               preferred_element_type=jnp.float32)


def _pre_body(h_ref, wa_ref, wb_ref, ha_ref, hb_ref):
    h = h_ref[...]
    ha_ref[...] = _dot(h, wa_ref[...])
    hb_ref[...] = _dot(h, wb_ref[...])


def _tc_pre(h, wa, wb):
    n = h.shape[0]
    grid = n // BN
    return pl.pallas_call(
        _pre_body,
        grid=(grid,),
        in_specs=[
            pl.BlockSpec((BN, 128), lambda i: (i, 0)),
            pl.BlockSpec((128, 128), lambda i: (0, 0)),
            pl.BlockSpec((128, 128), lambda i: (0, 0)),
        ],
        out_specs=[
            pl.BlockSpec((BN, 128), lambda i: (i, 0)),
            pl.BlockSpec((BN, 128), lambda i: (i, 0)),
        ],
        out_shape=[
            jax.ShapeDtypeStruct((n, 128), jnp.float32),
            jax.ShapeDtypeStruct((n, 128), jnp.float32),
        ],
    )(h, wa, wb)


def _edge_body(har_ref, hbc_ref, posr_ref, posc_ref,
               we1c_ref, be1_ref, we2_ref, be2_ref,
               wc1_ref, bc1_ref, wc2_ref,
               m_ref, trans_ref):
    dpos = posr_ref[...] - posc_ref[...]
    radial = jnp.sum(dpos * dpos, axis=-1, keepdims=True)
    x = har_ref[...] + hbc_ref[...] + radial * we1c_ref[...] + be1_ref[...]
    x = _silu(x)
    m = _silu(_dot(x, we2_ref[...]) + be2_ref[...])
    y = _silu(_dot(m, wc1_ref[...]) + bc1_ref[...])
    q = jnp.sum(y * wc2_ref[...], axis=-1, keepdims=True)
    lane = jax.lax.broadcasted_iota(jnp.int32, dpos.shape, 1)
    trans_ref[...] = dpos * q + jnp.where(lane == 3, 1.0, 0.0)
    m_ref[...] = m


def _tc_edge(har, hbc, posr, posc, we1c, be1, we2, be2, wc1, bc1, wc2row):
    e = har.shape[0]
    grid = e // BE
    w0 = pl.BlockSpec((1, 128), lambda i: (0, 0))
    return pl.pallas_call(
        _edge_body,
        grid=(grid,),
        in_specs=[
            pl.BlockSpec((BE, 128), lambda i: (i, 0)),
            pl.BlockSpec((BE, 128), lambda i: (i, 0)),
            pl.BlockSpec((BE, 16), lambda i: (i, 0)),
            pl.BlockSpec((BE, 16), lambda i: (i, 0)),
            w0, w0,
            pl.BlockSpec((128, 128), lambda i: (0, 0)), w0,
            pl.BlockSpec((128, 128), lambda i: (0, 0)), w0, w0,
        ],
        out_specs=[
            pl.BlockSpec((BE, 128), lambda i: (i, 0)),
            pl.BlockSpec((BE, 16), lambda i: (i, 0)),
        ],
        out_shape=[
            jax.ShapeDtypeStruct((e, 128), jnp.float32),
            jax.ShapeDtypeStruct((e, 16), jnp.float32),
        ],
    )(har, hbc, posr, posc, we1c, be1, we2, be2, wc1, bc1, wc2row)


def _node_body(h_ref, g_ref, vel_ref, pos_ref, agg_ref, fsum_ref,
               wn1h_ref, wn1a_ref, bn1_ref, wn2_ref, bn2_ref,
               ws1_ref, bs1_ref, ws2_ref, bs2_ref,
               wanx_ref, wbnx_ref,
               h2_ref, g2_ref, vel2_ref, pos2_ref, ha_ref, hb_ref, ldj_ref):
    i = pl.program_id(0)
    agg = agg_ref[0] + agg_ref[1]
    fsum = fsum_ref[0] + fsum_ref[1]
    cnt = jnp.maximum(fsum[:, 3:4], 1.0)
    lane = jax.lax.broadcasted_iota(jnp.int32, fsum.shape, 1)
    force = jnp.where(lane < 3, fsum, 0.0) / cnt * COORDS_WEIGHT
    h = h_ref[...]
    hf = _dot(_silu(_dot(h, wn1h_ref[...]) + _dot(agg, wn1a_ref[...])
                    + bn1_ref[...]), wn2_ref[...]) + bn2_ref[...]
    y = _silu(_dot(agg, ws1_ref[...]) + bs1_ref[...])
    scaling = jnp.tanh(jnp.sum(y * ws2_ref[...], axis=-1, keepdims=True)
                       + bs2_ref[...])
    vel2 = jnp.exp(scaling) * vel_ref[...] + force * DT
    pos2 = pos_ref[...] + vel2 * DT
    g2 = g_ref[...] + hf
    h2 = h + g2
    h2_ref[...] = h2
    g2_ref[...] = g2
    vel2_ref[...] = vel2
    pos2_ref[...] = pos2
    ha_ref[...] = _dot(h2, wanx_ref[...])
    hb_ref[...] = _dot(h2, wbnx_ref[...])

    @pl.when(i == 0)
    def _():
        ldj_ref[...] = jnp.zeros_like(ldj_ref)

    ldj_ref[0, 0] += jnp.sum(scaling)


def _tc_node(h, g, vel16, pos16, aggp, fp,
             wn1h, wn1a, bn1, wn2, bn2, ws1, bs1, ws2row, bs2, wanx, wbnx):
    n = h.shape[0]
    grid = n // BN
    w128 = pl.BlockSpec((128, 128), lambda i: (0, 0))
    w0 = pl.BlockSpec((1, 128), lambda i: (0, 0))
    return pl.pallas_call(
        _node_body,
        grid=(grid,),
        in_specs=[
            pl.BlockSpec((BN, 128), lambda i: (i, 0)),
            pl.BlockSpec((BN, 128), lambda i: (i, 0)),
            pl.BlockSpec((BN, 16), lambda i: (i, 0)),
            pl.BlockSpec((BN, 16), lambda i: (i, 0)),
            pl.BlockSpec((2, BN, 128), lambda i: (0, i, 0)),
            pl.BlockSpec((2, BN, 16), lambda i: (0, i, 0)),
            w128, w128, w0, w128, w0,
            w128, w0, w0, pl.BlockSpec((1, 1), lambda i: (0, 0)),
            w128, w128,
        ],
        out_specs=[
            pl.BlockSpec((BN, 128), lambda i: (i, 0)),
            pl.BlockSpec((BN, 128), lambda i: (i, 0)),
            pl.BlockSpec((BN, 16), lambda i: (i, 0)),
            pl.BlockSpec((BN, 16), lambda i: (i, 0)),
            pl.BlockSpec((BN, 128), lambda i: (i, 0)),
            pl.BlockSpec((BN, 128), lambda i: (i, 0)),
            pl.BlockSpec((1, 1), lambda i: (0, 0)),
        ],
        out_shape=[
            jax.ShapeDtypeStruct((n, 128), jnp.float32),
            jax.ShapeDtypeStruct((n, 128), jnp.float32),
            jax.ShapeDtypeStruct((n, 16), jnp.float32),
            jax.ShapeDtypeStruct((n, 16), jnp.float32),
            jax.ShapeDtypeStruct((n, 128), jnp.float32),
            jax.ShapeDtypeStruct((n, 128), jnp.float32),
            jax.ShapeDtypeStruct((1, 1), jnp.float32),
        ],
    )(h, g, vel16, pos16, aggp, fp,
      wn1h, wn1a, bn1, wn2, bn2, ws1, bs1, ws2row, bs2, wanx, wbnx)


def _gather_edges(ha, hb, pos16, row, col):
    return ha[row], hb[col], pos16[row], pos16[col]


def _scatter_edges(m, trans16, row, n):
    agg = jax.ops.segment_sum(m, row, num_segments=n)
    f = jax.ops.segment_sum(trans16, row, num_segments=n)
    z128 = jnp.zeros_like(agg)
    z16 = jnp.zeros_like(f)
    return jnp.stack([agg, z128]), jnp.stack([f, z16])


def kernel(h, pos, vel, g, edge_index, We1, be1, We2, be2, Wc1, bc1, Wc2,
           Wn1, bn1, Wn2, bn2, Ws1, bs1, Ws2, bs2):
    n = h.shape[0]
    row = edge_index[0]
    col = edge_index[1]
    pos16 = jnp.zeros((n, 16), jnp.float32).at[:, :3].set(pos)
    vel16 = jnp.zeros((n, 16), jnp.float32).at[:, :3].set(vel)

    ha, hb = _tc_pre(h, We1[0, :128], We1[0, 128:256])
    ldj = jnp.float32(0.0)
    for i in range(N_ITER):
        we1c = We1[i, 256:257]
        har, hbc, posr, posc = _gather_edges(ha, hb, pos16, row, col)
        m, trans16 = _tc_edge(har, hbc, posr, posc,
                              we1c, be1[i][None], We2[i], be2[i][None],
                              Wc1[i], bc1[i][None], Wc2[i, :, 0][None])
        aggp, fp = _scatter_edges(m, trans16, row, n)
        nx = min(i + 1, N_ITER - 1)
        h, g, vel16, pos16, ha, hb, ldj_i = _tc_node(
            h, g, vel16, pos16, aggp, fp,
            Wn1[i, :128], Wn1[i, 128:], bn1[i][None], Wn2[i], bn2[i][None],
            Ws1[i], bs1[i][None], Ws2[i, :, 0][None], bs2[i][None],
            We1[nx, :128], We1[nx, 128:256])
        ldj = ldj + ldj_i[0, 0]

    return (h, pos16[:, :3], vel16[:, :3], g, ldj)


# speedup vs baseline: 1.0023x; 1.0023x over previous
import functools

import jax
import jax.numpy as jnp
from jax import lax
from jax.experimental import pallas as pl
from jax.experimental.pallas import tpu as pltpu

N_ITER = 2
DT = 0.01
COORDS_WEIGHT = 1.0

BE = 512
BN = 1000

_HIGH = jax.lax.Precision.HIGHEST


def _silu(x):
    return x * jax.nn.sigmoid(x)


def _dot(a, b):
    return jax.lax.dot_general(a, b, (((1,), (0,)), ((), ())),
                               precision=_HIGH,
                               preferred_element_type=jnp.float32)


def _pre_body(h_ref, wa_ref, wb_ref, ha_ref, hb_ref):
    h = h_ref[...]
    ha_ref[...] = _dot(h, wa_ref[...])
    hb_ref[...] = _dot(h, wb_ref[...])


def _tc_pre(h, wa, wb):
    n = h.shape[0]
    grid = n // BN
    return pl.pallas_call(
        _pre_body,
        grid=(grid,),
        in_specs=[
            pl.BlockSpec((BN, 128), lambda i: (i, 0)),
            pl.BlockSpec((128, 128), lambda i: (0, 0)),
            pl.BlockSpec((128, 128), lambda i: (0, 0)),
        ],
        out_specs=[
            pl.BlockSpec((BN, 128), lambda i: (i, 0)),
            pl.BlockSpec((BN, 128), lambda i: (i, 0)),
        ],
        out_shape=[
            jax.ShapeDtypeStruct((n, 128), jnp.float32),
            jax.ShapeDtypeStruct((n, 128), jnp.float32),
        ],
    )(h, wa, wb)


def _edge_body(har_ref, hbc_ref, posr_ref, posc_ref,
               we1c_ref, be1_ref, we2_ref, be2_ref,
               wc1_ref, bc1_ref, wc2_ref,
               m_ref, trans_ref):
    dpos = posr_ref[...] - posc_ref[...]
    radial = jnp.sum(dpos * dpos, axis=-1, keepdims=True)
    x = har_ref[...] + hbc_ref[...] + radial * we1c_ref[...] + be1_ref[...]
    x = _silu(x)
    m = _silu(_dot(x, we2_ref[...]) + be2_ref[...])
    y = _silu(_dot(m, wc1_ref[...]) + bc1_ref[...])
    q = jnp.sum(y * wc2_ref[...], axis=-1, keepdims=True)
    lane = jax.lax.broadcasted_iota(jnp.int32, dpos.shape, 1)
    trans_ref[...] = dpos * q + jnp.where(lane == 3, 1.0, 0.0)
    m_ref[...] = m


def _tc_edge(har, hbc, posr, posc, we1c, be1, we2, be2, wc1, bc1, wc2row):
    e = har.shape[0]
    grid = e // BE
    w0 = pl.BlockSpec((1, 128), lambda i: (0, 0))
    return pl.pallas_call(
        _edge_body,
        grid=(grid,),
        in_specs=[
            pl.BlockSpec((BE, 128), lambda i: (i, 0)),
            pl.BlockSpec((BE, 128), lambda i: (i, 0)),
            pl.BlockSpec((BE, 16), lambda i: (i, 0)),
            pl.BlockSpec((BE, 16), lambda i: (i, 0)),
            w0, w0,
            pl.BlockSpec((128, 128), lambda i: (0, 0)), w0,
            pl.BlockSpec((128, 128), lambda i: (0, 0)), w0, w0,
        ],
        out_specs=[
            pl.BlockSpec((BE, 128), lambda i: (i, 0)),
            pl.BlockSpec((BE, 16), lambda i: (i, 0)),
        ],
        out_shape=[
            jax.ShapeDtypeStruct((e, 128), jnp.float32),
            jax.ShapeDtypeStruct((e, 16), jnp.float32),
        ],
    )(har, hbc, posr, posc, we1c, be1, we2, be2, wc1, bc1, wc2row)


def _node_body(h_ref, g_ref, vel_ref, pos_ref, agg_ref, fsum_ref,
               wn1h_ref, wn1a_ref, bn1_ref, wn2_ref, bn2_ref,
               ws1_ref, bs1_ref, ws2_ref, bs2_ref,
               wanx_ref, wbnx_ref,
               h2_ref, g2_ref, vel2_ref, pos2_ref, ha_ref, hb_ref, ldj_ref):
    i = pl.program_id(0)
    agg = agg_ref[0] + agg_ref[1]
    fsum = fsum_ref[0] + fsum_ref[1]
    cnt = jnp.maximum(fsum[:, 3:4], 1.0)
    lane = jax.lax.broadcasted_iota(jnp.int32, fsum.shape, 1)
    force = jnp.where(lane < 3, fsum, 0.0) / cnt * COORDS_WEIGHT
    h = h_ref[...]
    hf = _dot(_silu(_dot(h, wn1h_ref[...]) + _dot(agg, wn1a_ref[...])
                    + bn1_ref[...]), wn2_ref[...]) + bn2_ref[...]
    y = _silu(_dot(agg, ws1_ref[...]) + bs1_ref[...])
    scaling = jnp.tanh(jnp.sum(y * ws2_ref[...], axis=-1, keepdims=True)
                       + bs2_ref[...])
    vel2 = jnp.exp(scaling) * vel_ref[...] + force * DT
    pos2 = pos_ref[...] + vel2 * DT
    g2 = g_ref[...] + hf
    h2 = h + g2
    h2_ref[...] = h2
    g2_ref[...] = g2
    vel2_ref[...] = vel2
    pos2_ref[...] = pos2
    ha_ref[...] = _dot(h2, wanx_ref[...])
    hb_ref[...] = _dot(h2, wbnx_ref[...])

    @pl.when(i == 0)
    def _():
        ldj_ref[...] = jnp.zeros_like(ldj_ref)

    ldj_ref[...] = ldj_ref[...] + jnp.sum(scaling).reshape(1, 1)


def _tc_node(h, g, vel16, pos16, aggp, fp,
             wn1h, wn1a, bn1, wn2, bn2, ws1, bs1, ws2row, bs2, wanx, wbnx):
    n = h.shape[0]
    grid = n // BN
    w128 = pl.BlockSpec((128, 128), lambda i: (0, 0))
    w0 = pl.BlockSpec((1, 128), lambda i: (0, 0))
    return pl.pallas_call(
        _node_body,
        grid=(grid,),
        in_specs=[
            pl.BlockSpec((BN, 128), lambda i: (i, 0)),
            pl.BlockSpec((BN, 128), lambda i: (i, 0)),
            pl.BlockSpec((BN, 16), lambda i: (i, 0)),
            pl.BlockSpec((BN, 16), lambda i: (i, 0)),
            pl.BlockSpec((2, BN, 128), lambda i: (0, i, 0)),
            pl.BlockSpec((2, BN, 16), lambda i: (0, i, 0)),
            w128, w128, w0, w128, w0,
            w128, w0, w0, pl.BlockSpec((1, 1), lambda i: (0, 0)),
            w128, w128,
        ],
        out_specs=[
            pl.BlockSpec((BN, 128), lambda i: (i, 0)),
            pl.BlockSpec((BN, 128), lambda i: (i, 0)),
            pl.BlockSpec((BN, 16), lambda i: (i, 0)),
            pl.BlockSpec((BN, 16), lambda i: (i, 0)),
            pl.BlockSpec((BN, 128), lambda i: (i, 0)),
            pl.BlockSpec((BN, 128), lambda i: (i, 0)),
            pl.BlockSpec((1, 1), lambda i: (0, 0)),
        ],
        out_shape=[
            jax.ShapeDtypeStruct((n, 128), jnp.float32),
            jax.ShapeDtypeStruct((n, 128), jnp.float32),
            jax.ShapeDtypeStruct((n, 16), jnp.float32),
            jax.ShapeDtypeStruct((n, 16), jnp.float32),
            jax.ShapeDtypeStruct((n, 128), jnp.float32),
            jax.ShapeDtypeStruct((n, 128), jnp.float32),
            jax.ShapeDtypeStruct((1, 1), jnp.float32),
        ],
    )(h, g, vel16, pos16, aggp, fp,
      wn1h, wn1a, bn1, wn2, bn2, ws1, bs1, ws2row, bs2, wanx, wbnx)


def _gather_edges(ha, hb, pos16, row, col):
    return ha[row], hb[col], pos16[row], pos16[col]


def _scatter_edges(m, trans16, row, n):
    agg = jax.ops.segment_sum(m, row, num_segments=n)
    f = jax.ops.segment_sum(trans16, row, num_segments=n)
    z128 = jnp.zeros_like(agg)
    z16 = jnp.zeros_like(f)
    return jnp.stack([agg, z128]), jnp.stack([f, z16])


def kernel(h, pos, vel, g, edge_index, We1, be1, We2, be2, Wc1, bc1, Wc2,
           Wn1, bn1, Wn2, bn2, Ws1, bs1, Ws2, bs2):
    n = h.shape[0]
    row = edge_index[0]
    col = edge_index[1]
    pos16 = jnp.zeros((n, 16), jnp.float32).at[:, :3].set(pos)
    vel16 = jnp.zeros((n, 16), jnp.float32).at[:, :3].set(vel)

    ha, hb = _tc_pre(h, We1[0, :128], We1[0, 128:256])
    ldj = jnp.float32(0.0)
    for i in range(N_ITER):
        we1c = We1[i, 256:257]
        har, hbc, posr, posc = _gather_edges(ha, hb, pos16, row, col)
        m, trans16 = _tc_edge(har, hbc, posr, posc,
                              we1c, be1[i][None], We2[i], be2[i][None],
                              Wc1[i], bc1[i][None], Wc2[i, :, 0][None])
        aggp, fp = _scatter_edges(m, trans16, row, n)
        nx = min(i + 1, N_ITER - 1)
        h, g, vel16, pos16, ha, hb, ldj_i = _tc_node(
            h, g, vel16, pos16, aggp, fp,
            Wn1[i, :128], Wn1[i, 128:], bn1[i][None], Wn2[i], bn2[i][None],
            Ws1[i], bs1[i][None], Ws2[i, :, 0][None], bs2[i][None],
            We1[nx, :128], We1[nx, 128:256])
        ldj = ldj + ldj_i[0, 0]

    return (h, pos16[:, :3], vel16[:, :3], g, ldj)


# speedup vs baseline: 2.9768x; 2.9699x over previous
import functools

import jax
import jax.numpy as jnp
from jax import lax
from jax.experimental import pallas as pl
from jax.experimental.pallas import tpu as pltpu

N_ITER = 2
DT = 0.01
COORDS_WEIGHT = 1.0

BE = 512
BN = 1000

def _silu(x):
    return x * (0.5 * jnp.tanh(0.5 * x) + 0.5)


def _dot(a, b):
    return jax.lax.dot_general(a, b, (((1,), (0,)), ((), ())),
                               preferred_element_type=jnp.float32)


def _rowdot(x, wrow):
    xb = x.astype(jnp.bfloat16).astype(jnp.float32)
    wb = wrow.astype(jnp.bfloat16).astype(jnp.float32)
    return jnp.sum(xb * wb, axis=-1, keepdims=True)


def _pre_body(h_ref, wa_ref, wb_ref, ha_ref, hb_ref):
    h = h_ref[...]
    ha_ref[...] = _dot(h, wa_ref[...])
    hb_ref[...] = _dot(h, wb_ref[...])


def _tc_pre(h, wa, wb):
    n = h.shape[0]
    grid = n // BN
    return pl.pallas_call(
        _pre_body,
        grid=(grid,),
        in_specs=[
            pl.BlockSpec((BN, 128), lambda i: (i, 0)),
            pl.BlockSpec((128, 128), lambda i: (0, 0)),
            pl.BlockSpec((128, 128), lambda i: (0, 0)),
        ],
        out_specs=[
            pl.BlockSpec((BN, 128), lambda i: (i, 0)),
            pl.BlockSpec((BN, 128), lambda i: (i, 0)),
        ],
        out_shape=[
            jax.ShapeDtypeStruct((n, 128), jnp.float32),
            jax.ShapeDtypeStruct((n, 128), jnp.float32),
        ],
    )(h, wa, wb)


def _edge_body(har_ref, hbc_ref, posr_ref, posc_ref,
               we1c_ref, be1_ref, we2_ref, be2_ref,
               wc1_ref, bc1_ref, wc2_ref,
               m_ref, trans_ref):
    dpos = posr_ref[...] - posc_ref[...]
    radial = jnp.sum(dpos * dpos, axis=-1, keepdims=True)
    rb = radial.astype(jnp.bfloat16).astype(jnp.float32)
    wb = we1c_ref[...].astype(jnp.bfloat16).astype(jnp.float32)
    x = (har_ref[...] + hbc_ref[...]) + rb * wb + be1_ref[...]
    x = _silu(x)
    m = _silu(_dot(x, we2_ref[...]) + be2_ref[...])
    y = _silu(_dot(m, wc1_ref[...]) + bc1_ref[...])
    q = _dot(y, wc2_ref[...])
    lane = jax.lax.broadcasted_iota(jnp.int32, dpos.shape, 1)
    trans_ref[...] = dpos * q + jnp.where(lane == 3, 1.0, 0.0)
    m_ref[...] = m


def _tc_edge(har, hbc, posr, posc, we1c, be1, we2, be2, wc1, bc1, wc2row):
    e = har.shape[0]
    grid = e // BE
    w0 = pl.BlockSpec((1, 128), lambda i: (0, 0))
    return pl.pallas_call(
        _edge_body,
        grid=(grid,),
        in_specs=[
            pl.BlockSpec((BE, 128), lambda i: (i, 0)),
            pl.BlockSpec((BE, 128), lambda i: (i, 0)),
            pl.BlockSpec((BE, 128), lambda i: (i, 0)),
            pl.BlockSpec((BE, 128), lambda i: (i, 0)),
            w0, w0,
            pl.BlockSpec((128, 128), lambda i: (0, 0)), w0,
            pl.BlockSpec((128, 128), lambda i: (0, 0)), w0,
            pl.BlockSpec((128, 1), lambda i: (0, 0)),
        ],
        out_specs=[
            pl.BlockSpec((BE, 128), lambda i: (i, 0)),
            pl.BlockSpec((BE, 128), lambda i: (i, 0)),
        ],
        out_shape=[
            jax.ShapeDtypeStruct((e, 128), jnp.float32),
            jax.ShapeDtypeStruct((e, 128), jnp.float32),
        ],
    )(har, hbc, posr, posc, we1c, be1, we2, be2, wc1, bc1, wc2row)


def _node_body(h_ref, g_ref, vel_ref, pos_ref, agg_ref, fsum_ref,
               wn1h_ref, wn1a_ref, bn1_ref, wn2_ref, bn2_ref,
               ws1_ref, bs1_ref, ws2_ref, bs2_ref,
               wanx_ref, wbnx_ref,
               h2_ref, g2_ref, vel2_ref, pos2_ref, ha_ref, hb_ref, ldj_ref):
    i = pl.program_id(0)
    agg = agg_ref[0] + agg_ref[1]
    fsum = fsum_ref[0] + fsum_ref[1]
    cnt = jnp.maximum(fsum[:, 3:4], 1.0)
    lane = jax.lax.broadcasted_iota(jnp.int32, fsum.shape, 1)
    force = jnp.where(lane < 3, fsum, 0.0) / cnt * COORDS_WEIGHT
    h = h_ref[...]
    hf = _dot(_silu(_dot(h, wn1h_ref[...]) + _dot(agg, wn1a_ref[...])
                    + bn1_ref[...]), wn2_ref[...]) + bn2_ref[...]
    y = _silu(_dot(agg, ws1_ref[...]) + bs1_ref[...])
    scaling = jnp.tanh(_dot(y, ws2_ref[...]) + bs2_ref[...])
    vel2 = jnp.exp(scaling) * vel_ref[...] + force * DT
    pos2 = pos_ref[...] + vel2 * DT
    g2 = g_ref[...] + hf
    h2 = h + g2
    h2_ref[...] = h2
    g2_ref[...] = g2
    vel2_ref[...] = vel2
    pos2_ref[...] = pos2
    ha_ref[...] = _dot(h2, wanx_ref[...])
    hb_ref[...] = _dot(h2, wbnx_ref[...])

    @pl.when(i == 0)
    def _():
        ldj_ref[...] = jnp.zeros_like(ldj_ref)

    ldj_ref[...] = ldj_ref[...] + jnp.sum(scaling).reshape(1, 1)


def _tc_node(h, g, vel16, pos16, aggp, fp,
             wn1h, wn1a, bn1, wn2, bn2, ws1, bs1, ws2row, bs2, wanx, wbnx):
    n = h.shape[0]
    grid = n // BN
    w128 = pl.BlockSpec((128, 128), lambda i: (0, 0))
    w0 = pl.BlockSpec((1, 128), lambda i: (0, 0))
    return pl.pallas_call(
        _node_body,
        grid=(grid,),
        in_specs=[
            pl.BlockSpec((BN, 128), lambda i: (i, 0)),
            pl.BlockSpec((BN, 128), lambda i: (i, 0)),
            pl.BlockSpec((BN, 128), lambda i: (i, 0)),
            pl.BlockSpec((BN, 128), lambda i: (i, 0)),
            pl.BlockSpec((2, BN, 128), lambda i: (0, i, 0)),
            pl.BlockSpec((2, BN, 128), lambda i: (0, i, 0)),
            w128, w128, w0, w128, w0,
            w128, w0, pl.BlockSpec((128, 1), lambda i: (0, 0)),
            pl.BlockSpec((1, 1), lambda i: (0, 0)),
            w128, w128,
        ],
        out_specs=[
            pl.BlockSpec((BN, 128), lambda i: (i, 0)),
            pl.BlockSpec((BN, 128), lambda i: (i, 0)),
            pl.BlockSpec((BN, 128), lambda i: (i, 0)),
            pl.BlockSpec((BN, 128), lambda i: (i, 0)),
            pl.BlockSpec((BN, 128), lambda i: (i, 0)),
            pl.BlockSpec((BN, 128), lambda i: (i, 0)),
            pl.BlockSpec((1, 1), lambda i: (0, 0)),
        ],
        out_shape=[
            jax.ShapeDtypeStruct((n, 128), jnp.float32),
            jax.ShapeDtypeStruct((n, 128), jnp.float32),
            jax.ShapeDtypeStruct((n, 128), jnp.float32),
            jax.ShapeDtypeStruct((n, 128), jnp.float32),
            jax.ShapeDtypeStruct((n, 128), jnp.float32),
            jax.ShapeDtypeStruct((n, 128), jnp.float32),
            jax.ShapeDtypeStruct((1, 1), jnp.float32),
        ],
    )(h, g, vel16, pos16, aggp, fp,
      wn1h, wn1a, bn1, wn2, bn2, ws1, bs1, ws2row, bs2, wanx, wbnx)


NC = 2
NS = 16
BS = 80
_sc_mesh = None


def _get_sc_mesh():
    global _sc_mesh
    if _sc_mesh is None:
        from jax.experimental.pallas import tpu_sc as plsc
        _sc_mesh = plsc.VectorSubcoreMesh(core_axis_name="c",
                                          subcore_axis_name="s")
    return _sc_mesh


def _gather_body(nblk, ha_hbm, hb_hbm, pos_hbm, row_hbm, col_hbm,
                 har_hbm, hbc_hbm, posr_hbm, posc_hbm,
                 row_v, col_v, bufa, bufb, bufp, bufq, sem):
    wid = lax.axis_index("s") * NC + lax.axis_index("c")
    ebase = wid * nblk * BS
    pltpu.sync_copy(row_hbm.at[wid], row_v)
    pltpu.sync_copy(col_hbm.at[wid], col_v)

    def body(b, _):
        e0 = ebase + b * BS
        ca = pltpu.async_copy(ha_hbm.at[row_v.at[b]], bufa, sem)
        cb = pltpu.async_copy(hb_hbm.at[col_v.at[b]], bufb, sem)
        cp = pltpu.async_copy(pos_hbm.at[row_v.at[b]], bufp, sem)
        cq = pltpu.async_copy(pos_hbm.at[col_v.at[b]], bufq, sem)
        ca.wait(); cb.wait(); cp.wait(); cq.wait()
        pltpu.sync_copy(bufa, har_hbm.at[pl.ds(e0, BS)])
        pltpu.sync_copy(bufb, hbc_hbm.at[pl.ds(e0, BS)])
        pltpu.sync_copy(bufp, posr_hbm.at[pl.ds(e0, BS)])
        pltpu.sync_copy(bufq, posc_hbm.at[pl.ds(e0, BS)])
        return ()

    lax.fori_loop(0, nblk, body, (), unroll=False)


def _gather_edges(ha, hb, pos128, row3d, col3d):
    nw, nblk, _ = row3d.shape
    e = nw * nblk * BS
    f = pl.kernel(
        functools.partial(_gather_body, nblk),
        out_type=[
            jax.ShapeDtypeStruct((e, 128), jnp.float32),
            jax.ShapeDtypeStruct((e, 128), jnp.float32),
            jax.ShapeDtypeStruct((e, 128), jnp.float32),
            jax.ShapeDtypeStruct((e, 128), jnp.float32),
        ],
        mesh=_get_sc_mesh(),
        scratch_types=[
            pltpu.VMEM((nblk, BS), jnp.int32),
            pltpu.VMEM((nblk, BS), jnp.int32),
            pltpu.VMEM((BS, 128), jnp.float32),
            pltpu.VMEM((BS, 128), jnp.float32),
            pltpu.VMEM((BS, 128), jnp.float32),
            pltpu.VMEM((BS, 128), jnp.float32),
            pltpu.SemaphoreType.DMA,
        ],
    )
    return f(ha, hb, pos128, row3d, col3d)


def _scatter_body(nblk, n, m_hbm, row_hbm, z128_hbm,
                  aggp_hbm, row_v, mbuf, agg_sp):
    from jax.experimental.pallas import tpu_sc as plsc
    cid = lax.axis_index("c")
    sid = lax.axis_index("s")
    wid = sid * NC + cid
    ebase = wid * nblk * BS
    pltpu.sync_copy(row_hbm.at[wid], row_v)

    @pl.when(sid == 0)
    def _():
        pltpu.sync_copy(z128_hbm, agg_sp)

    plsc.subcore_barrier()

    def body(b, _):
        e0 = ebase + b * BS
        pltpu.sync_copy(m_hbm.at[pl.ds(e0, BS)], mbuf)
        pltpu.sync_copy(mbuf, agg_sp.at[row_v.at[b]], add=True)
        return ()

    lax.fori_loop(0, nblk, body, (), unroll=False)
    plsc.subcore_barrier()
    stripe = 8 * (n // (8 * NS))
    last = n - (NS - 1) * stripe
    r0 = sid * stripe

    @pl.when(sid < NS - 1)
    def _():
        pltpu.sync_copy(agg_sp.at[pl.ds(r0, stripe)],
                        aggp_hbm.at[cid, pl.ds(r0, stripe)])

    @pl.when(sid == NS - 1)
    def _():
        pltpu.sync_copy(agg_sp.at[pl.ds((NS - 1) * stripe, last)],
                        aggp_hbm.at[cid, pl.ds((NS - 1) * stripe, last)])


def _scatter_edges(m, row3d, z128):
    n = z128.shape[0]
    _, nblk, _ = row3d.shape
    f = pl.kernel(
        functools.partial(_scatter_body, nblk, n),
        out_type=jax.ShapeDtypeStruct((NC, n, 128), jnp.float32),
        mesh=_get_sc_mesh(),
        scratch_types=[
            pltpu.VMEM((nblk, BS), jnp.int32),
            pltpu.VMEM((BS, 128), jnp.float32),
            pltpu.VMEM_SHARED((n, 128), jnp.float32),
        ],
    )
    return f(m, row3d, z128)


def kernel(h, pos, vel, g, edge_index, We1, be1, We2, be2, Wc1, bc1, Wc2,
           Wn1, bn1, Wn2, bn2, Ws1, bs1, Ws2, bs2):
    n = h.shape[0]
    e = edge_index.shape[1]
    nw = NC * NS
    row3d = edge_index[0].reshape(nw, e // (nw * BS), BS)
    col3d = edge_index[1].reshape(nw, e // (nw * BS), BS)
    pos128 = jnp.zeros((n, 128), jnp.float32).at[:, :3].set(pos)
    vel128 = jnp.zeros((n, 128), jnp.float32).at[:, :3].set(vel)
    z128 = jnp.zeros((n, 128), jnp.float32)

    ha, hb = _tc_pre(h, We1[0, :128], We1[0, 128:256])
    ldj = jnp.float32(0.0)
    for i in range(N_ITER):
        we1c = We1[i, 256:257]
        har, hbc, posr, posc = _gather_edges(ha, hb, pos128, row3d, col3d)
        m, trans = _tc_edge(har, hbc, posr, posc,
                            we1c, be1[i][None], We2[i], be2[i][None],
                            Wc1[i], bc1[i][None], Wc2[i])
        aggp = _scatter_edges(m, row3d, z128)
        fp = _scatter_edges(trans, row3d, z128)
        nx = min(i + 1, N_ITER - 1)
        h, g, vel128, pos128, ha, hb, ldj_i = _tc_node(
            h, g, vel128, pos128, aggp, fp,
            Wn1[i, :128], Wn1[i, 128:], bn1[i][None], Wn2[i], bn2[i][None],
            Ws1[i], bs1[i][None], Ws2[i], bs2[i][None],
            We1[nx, :128], We1[nx, 128:256])
        ldj = ldj + ldj_i[0, 0]

    return (h, pos128[:, :3], vel128[:, :3], g, ldj)
